# baseline (speedup 1.0000x reference)
import functools

import jax
import jax.numpy as jnp
from jax import lax
from jax.experimental import pallas as pl
from jax.experimental.pallas import tpu as pltpu
from jax.experimental.pallas import tpu_sc as plsc

B = 1024
L = 200
DIM = 64
N = B * L
NW = 32
CHUNK = 512
NCHUNK = N // CHUNK
GSUB = 128


def _gather_body(ids_ref, table2_ref, out_ref, idx_v, idx2_v, rows_v, sem):
    wid = lax.axis_index("s") * 2 + lax.axis_index("c")
    lane = lax.iota(jnp.int32, 16)

    def do_chunk(c, _):
        base = c * CHUNK
        pltpu.sync_copy(ids_ref.at[pl.ds(base, CHUNK)], idx_v)

        def halve(i, _):
            idx2_v[pl.ds(i * 16, 16)] = lax.shift_right_logical(
                idx_v[pl.ds(i * 16, 16)], 1)
            return _
        lax.fori_loop(0, CHUNK // 16, halve, 0)

        for k in range(CHUNK // GSUB):
            pltpu.async_copy(
                table2_ref.at[idx2_v.at[pl.ds(k * GSUB, GSUB)]],
                rows_v.at[pl.ds(k * GSUB, GSUB)], sem).wait()

        def select(g, _):
            rowi = g * 16 + lane
            colb = (idx_v[pl.ds(g * 16, 16)] & 1) * DIM

            def feat(f, _):
                v = plsc.load_gather(rows_v, [rowi, colb + f])
                plsc.store_scatter(rows_v, [rowi, jnp.full((16,), f, jnp.int32)], v)
                return _
            lax.fori_loop(0, DIM, feat, 0)
            return _
        lax.fori_loop(0, CHUNK // 16, select, 0)

        pltpu.sync_copy(rows_v, out_ref.at[pl.ds(base, CHUNK)])
        return _

    nfull = NCHUNK // NW
    lax.fori_loop(0, nfull, lambda t, _: do_chunk(t * NW + wid, _), 0)
    rem = NCHUNK - nfull * NW

    @pl.when(wid < rem)
    def _():
        do_chunk(nfull * NW + wid, 0)


@jax.jit
def _sc_gather(ids_flat, table2):
    mesh = plsc.VectorSubcoreMesh(core_axis_name="c", subcore_axis_name="s")
    return pl.kernel(
        _gather_body,
        out_type=jax.ShapeDtypeStruct((N, 128), jnp.float32),
        mesh=mesh,
        scratch_types=[
            pltpu.VMEM((CHUNK,), jnp.int32),
            pltpu.VMEM((CHUNK,), jnp.int32),
            pltpu.VMEM((CHUNK, 128), jnp.float32),
            pltpu.SemaphoreType.DMA,
        ],
        compiler_params=pltpu.CompilerParams(
            use_tc_tiling_on_sc=True, needs_layout_passes=False),
    )(ids_flat, table2)


def kernel(input_ids, attention_mask, table):
    ids_flat = input_ids.T.reshape(N).astype(jnp.int32)
    table2 = table.reshape(500000, 128)
    out128 = _sc_gather(ids_flat, table2)
    sel = out128[:, :DIM]
    tok = sel.reshape(L, B, DIM).transpose(1, 0, 2)
    return (tok, attention_mask)
